# baseline (speedup 1.0000x reference)
import functools
import math

import jax
import jax.numpy as jnp
from jax import lax
from jax.experimental import pallas as pl
from jax.experimental.pallas import tpu as pltpu
from jax.experimental.pallas import tpu_sc as plsc

NC = 2
NS = 16
NW = NC * NS
LANES = 16

N = 10000
E = 320000
D = 128
DK = 32
DE = 16
H = 64
L = 3

EW = E // NW
C = 80
NCH = EW // C
EXP_T = 10
EXP_R = N // EXP_T
ZR = 40
GROUPS = C // LANES

_MESH = dict(core_axis_name="c", subcore_axis_name="s", num_cores=NC,
             num_subcores=NS)


def _wid():
    return lax.axis_index("s") * NC + lax.axis_index("c")


def _prep_scales(ef, dist, Wkg, Wr1k, br1k, Wr2k, br2k, Wvg, Wr1v, br1v,
                 Wr2v, br2v):
    BE = 4000

    def body(ef_ref, dist_ref, Wkg_ref, Wr1k_ref, br1k_ref, Wr2k_ref,
             br2k_ref, Wvg_ref, Wr1v_ref, br1v_ref, Wr2v_ref, br2v_ref,
             ks_ref, vs_ref):
        efb = ef_ref[...]
        db = dist_ref[...]
        tk = jnp.tanh(db * Wr1k_ref[0] + br1k_ref[0])
        rk = jnp.dot(tk, Wr2k_ref[0], preferred_element_type=jnp.float32)
        rk = rk + br2k_ref[0]
        tv = jnp.tanh(db * Wr1v_ref[0] + br1v_ref[0])
        rv = jnp.dot(tv, Wr2v_ref[0], preferred_element_type=jnp.float32)
        rv = rv + br2v_ref[0]
        ks_ref[0] = jnp.dot(efb, Wkg_ref[0],
                            preferred_element_type=jnp.float32) * rk
        vs_ref[0] = jnp.dot(efb, Wvg_ref[0],
                            preferred_element_type=jnp.float32) * rv

    grid = (L, E // BE)
    out = pl.pallas_call(
        body,
        grid=grid,
        in_specs=[
            pl.BlockSpec((BE, DE), lambda l, b: (b, 0)),
            pl.BlockSpec((BE, 1), lambda l, b: (b, 0)),
            pl.BlockSpec((1, DE, DK), lambda l, b: (l, 0, 0)),
            pl.BlockSpec((1, 1, H), lambda l, b: (l, 0, 0)),
            pl.BlockSpec((1, 1, H), lambda l, b: (l, 0, 0)),
            pl.BlockSpec((1, H, DK), lambda l, b: (l, 0, 0)),
            pl.BlockSpec((1, 1, DK), lambda l, b: (l, 0, 0)),
            pl.BlockSpec((1, DE, D), lambda l, b: (l, 0, 0)),
            pl.BlockSpec((1, 1, H), lambda l, b: (l, 0, 0)),
            pl.BlockSpec((1, 1, H), lambda l, b: (l, 0, 0)),
            pl.BlockSpec((1, H, D), lambda l, b: (l, 0, 0)),
            pl.BlockSpec((1, 1, D), lambda l, b: (l, 0, 0)),
        ],
        out_specs=[
            pl.BlockSpec((1, BE, DK), lambda l, b: (l, b, 0)),
            pl.BlockSpec((1, BE, D), lambda l, b: (l, b, 0)),
        ],
        out_shape=[
            jax.ShapeDtypeStruct((L, E, DK), jnp.float32),
            jax.ShapeDtypeStruct((L, E, D), jnp.float32),
        ],
    )(ef, dist, Wkg, Wr1k, br1k[:, None, :], Wr2k, br2k[:, None, :],
      Wvg, Wr1v, br1v[:, None, :], Wr2v, br2v[:, None, :])
    return out


def _proj(x, Wq_l, Wkf_l, Wvf_l, Wvs_l):
    BN = 2000

    def body(x_ref, wq_ref, wkf_ref, wvf_ref, wvs_ref, q_ref, xk_ref,
             xv_ref, vs_ref):
        xb = x_ref[...]
        q_ref[...] = jnp.dot(xb, wq_ref[...],
                             preferred_element_type=jnp.float32)
        xk_ref[...] = jnp.dot(xb, wkf_ref[...],
                              preferred_element_type=jnp.float32)
        xv_ref[...] = jnp.dot(xb, wvf_ref[...],
                              preferred_element_type=jnp.float32)
        vs_ref[...] = jnp.dot(xb, wvs_ref[...],
                              preferred_element_type=jnp.float32)

    return pl.pallas_call(
        body,
        grid=(N // BN,),
        in_specs=[
            pl.BlockSpec((BN, D), lambda b: (b, 0)),
            pl.BlockSpec((D, DK), lambda b: (0, 0)),
            pl.BlockSpec((D, DK), lambda b: (0, 0)),
            pl.BlockSpec((D, D), lambda b: (0, 0)),
            pl.BlockSpec((D, D), lambda b: (0, 0)),
        ],
        out_specs=[
            pl.BlockSpec((BN, DK), lambda b: (b, 0)),
            pl.BlockSpec((BN, DK), lambda b: (b, 0)),
            pl.BlockSpec((BN, D), lambda b: (b, 0)),
            pl.BlockSpec((BN, D), lambda b: (b, 0)),
        ],
        out_shape=[
            jax.ShapeDtypeStruct((N, DK), jnp.float32),
            jax.ShapeDtypeStruct((N, DK), jnp.float32),
            jax.ShapeDtypeStruct((N, D), jnp.float32),
            jax.ShapeDtypeStruct((N, D), jnp.float32),
        ],
    )(x, Wq_l, Wkf_l, Wvf_l, Wvs_l)


@functools.cache
def _make_sc_pass1():
    return functools.partial(
        pl.kernel,
        out_type=[
            jax.ShapeDtypeStruct((NW, NCH, C), jnp.float32),
            jax.ShapeDtypeStruct((NW, N), jnp.float32),
        ],
        mesh=plsc.VectorSubcoreMesh(**_MESH),
        scratch_types=[
            pltpu.VMEM((N,), jnp.float32),
            pltpu.VMEM((NCH, C), jnp.int32),
            pltpu.VMEM((NCH, C), jnp.int32),
            pltpu.VMEM((2, C, DK), jnp.float32),
            pltpu.VMEM((2, C, DK), jnp.float32),
            pltpu.VMEM((2, C, DK), jnp.float32),
            pltpu.VMEM((2, C), jnp.float32),
            pltpu.VMEM((LANES,), jnp.int32),
            pltpu.VMEM((LANES,), jnp.float32),
            pltpu.SemaphoreType.DMA,
            pltpu.SemaphoreType.DMA,
            pltpu.SemaphoreType.DMA,
            pltpu.SemaphoreType.DMA,
        ],
        compiler_params=pltpu.CompilerParams(use_tc_tiling_on_sc=False,
                                             needs_layout_passes=False),
    )(_sc_pass1_body)


def _sc_pass1_body(src_hbm, dst_hbm, xk_hbm, q_hbm, ks_hbm,
                   lg_hbm, pmax_hbm,
                   marr, srcall, dstall, kbuf, qbuf, ksbuf, lgw, k16, v16,
                   sem_in0, sem_in1, sem_lg0, sem_lg1):
    wid = _wid()
    base = wid * EW
    sems = (sem_in0, sem_in1)
    sem_lg = (sem_lg0, sem_lg1)
    neg = jnp.full((LANES,), -1e30, dtype=jnp.float32)
    iota = lax.iota(jnp.int32, LANES)
    scale = 1.0 / math.sqrt(float(DK))

    pltpu.sync_copy(src_hbm.at[wid], srcall)
    pltpu.sync_copy(dst_hbm.at[wid], dstall)

    def init(i, carry):
        marr[pl.ds(i * LANES, LANES)] = neg
        return carry

    lax.fori_loop(0, N // LANES, init, 0)

    def fire(j, b):
        off = base + j * C
        pltpu.async_copy(xk_hbm.at[srcall.at[j]], kbuf.at[b], sems[b])
        pltpu.async_copy(q_hbm.at[dstall.at[j]], qbuf.at[b], sems[b])
        pltpu.async_copy(ks_hbm.at[pl.ds(off, C)], ksbuf.at[b], sems[b])

    def drain_in(b):
        pltpu.make_async_copy(xk_hbm.at[pl.ds(0, C)], kbuf.at[b],
                              sems[b]).wait()
        pltpu.make_async_copy(q_hbm.at[pl.ds(0, C)], qbuf.at[b],
                              sems[b]).wait()
        pltpu.make_async_copy(ks_hbm.at[pl.ds(0, C)], ksbuf.at[b],
                              sems[b]).wait()

    def compute(j, b):
        @pl.when(j >= 2)
        def _():
            pltpu.make_async_copy(lgw.at[b], lg_hbm.at[wid, 0],
                                  sem_lg[b]).wait()

        for g in range(GROUPS):
            rows = iota + (g * LANES)
            acc = jnp.zeros((LANES,), dtype=jnp.float32)
            for d in range(DK):
                cols = jnp.full((LANES,), d, dtype=jnp.int32)
                a = plsc.load_gather(kbuf.at[b], [rows, cols])
                kk = plsc.load_gather(ksbuf.at[b], [rows, cols])
                qq = plsc.load_gather(qbuf.at[b], [rows, cols])
                acc = acc + a * kk * qq
            val0 = acc * scale
            lgw[b, pl.ds(g * LANES, LANES)] = val0
            dvec = dstall[j, pl.ds(g * LANES, LANES)]
            ksrt, vsrt = plsc.sort_key_val(dvec, val0)
            k16[...] = ksrt
            val = vsrt
            for sh in (1, 2, 4, 8):
                v16[...] = val
                idx = jnp.maximum(iota - sh, 0)
                kprev = plsc.load_gather(k16, [idx])
                vprev = plsc.load_gather(v16, [idx])
                same = (kprev == ksrt) & (iota >= sh)
                val = jnp.where(same, jnp.maximum(val, vprev), val)
            knext = plsc.load_gather(k16, [jnp.minimum(iota + 1, LANES - 1)])
            is_last = (knext != ksrt) | (iota == LANES - 1)
            old = plsc.load_gather(marr, [ksrt])
            plsc.store_scatter(marr, [ksrt], jnp.maximum(old, val),
                               mask=is_last)
        pltpu.async_copy(lgw.at[b], lg_hbm.at[wid, j], sem_lg[b])

    fire(0, 0)

    def outer(t, carry):
        for b in range(2):
            j = 2 * t + b

            @pl.when(j + 1 < NCH)
            def _():
                fire(j + 1, b ^ 1)

            drain_in(b)
            compute(j, b)
        return carry

    lax.fori_loop(0, NCH // 2, outer, 0)
    jt = NCH - 1
    drain_in(jt % 2)
    compute(jt, jt % 2)
    for b in range(2):
        pltpu.make_async_copy(lgw.at[b], lg_hbm.at[wid, 0],
                              sem_lg[b]).wait()

    pltpu.sync_copy(marr, pmax_hbm.at[wid])


def _tc_mreduce(pmax):
    def body(p_ref, m_ref, sw_ref):
        m = jnp.max(p_ref[...], axis=1, keepdims=True)
        m = jnp.maximum(m, 1.0)
        m_ref[...] = m
        sw_ref[...] = jnp.exp(1.0 - m)

    return pl.pallas_call(
        body,
        out_shape=[
            jax.ShapeDtypeStruct((N, 1), jnp.float32),
            jax.ShapeDtypeStruct((N, 1), jnp.float32),
        ],
    )(pmax)


@functools.cache
def _make_sc_pass2():
    return functools.partial(
        pl.kernel,
        out_type=[
            jax.ShapeDtypeStruct((NW, 1, N), jnp.float32),
            jax.ShapeDtypeStruct((NC, N, D), jnp.float32),
        ],
        mesh=plsc.VectorSubcoreMesh(**_MESH),
        scratch_types=[
            pltpu.VMEM_SHARED((N, D), jnp.float32),
            pltpu.VMEM((N,), jnp.float32),
            pltpu.VMEM((1, N), jnp.float32),
            pltpu.VMEM((4, C), jnp.int32),
            pltpu.VMEM((4, C), jnp.int32),
            pltpu.VMEM((4, C), jnp.float32),
            pltpu.VMEM((C,), jnp.float32),
            pltpu.VMEM((2, C, D), jnp.float32),
            pltpu.VMEM((C // 2, D), jnp.float32),
            pltpu.SemaphoreType.DMA,
            pltpu.SemaphoreType.DMA,
            pltpu.SemaphoreType.DMA,
            pltpu.SemaphoreType.DMA,
            pltpu.SemaphoreType.DMA,
            pltpu.SemaphoreType.DMA,
            pltpu.SemaphoreType.DMA,
            pltpu.SemaphoreType.DMA,
        ],
        compiler_params=pltpu.CompilerParams(needs_layout_passes=False),
    )(_sc_pass2_body)


def _sc_pass2_body(src_hbm, dst_hbm, lg_hbm, m_hbm, xv_hbm, vs_hbm,
                   pden_hbm, pnum_hbm,
                   sharr, marrm, darr, isrc, idst2, lbuf, ebuf, vbuf,
                   vsbuf, sem_g0, sem_g1, sem_s0, sem_s1, sem_i0, sem_i1,
                   sem_i2, sem_i3):
    cid = lax.axis_index("c")
    sid = lax.axis_index("s")
    wid = _wid()
    base = wid * EW
    zero = jnp.zeros((LANES,), dtype=jnp.float32)

    pltpu.sync_copy(m_hbm, marrm)

    def zinit(i, carry):
        r = i // (D // LANES)
        col = (i % (D // LANES)) * LANES
        vsbuf[r, pl.ds(col, LANES)] = zero
        return carry

    lax.fori_loop(0, ZR * (D // LANES), zinit, 0)

    def dinit(i, carry):
        darr[0, pl.ds(i * LANES, LANES)] = zero
        return carry

    lax.fori_loop(0, N // LANES, dinit, 0)

    @pl.when(sid < EXP_T)
    def _zero():
        for p in range(EXP_R // ZR):
            pltpu.sync_copy(vsbuf, sharr.at[pl.ds(sid * EXP_R + p * ZR, ZR)])

    plsc.subcore_barrier()

    sem_i = (sem_i0, sem_i1, sem_i2, sem_i3)
    sem_g = (sem_g0, sem_g1)
    sem_s = (sem_s0, sem_s1)

    def load_idx(j, b):
        off = base + j * C
        pltpu.async_copy(src_hbm.at[pl.ds(off, C)], isrc.at[b], sem_i[b])
        pltpu.async_copy(dst_hbm.at[pl.ds(off, C)], idst2.at[b], sem_i[b])
        pltpu.async_copy(lg_hbm.at[pl.ds(off, C)], lbuf.at[b], sem_i[b])

    def drain_idx(b):
        pltpu.make_async_copy(src_hbm.at[pl.ds(0, C)], isrc.at[b],
                              sem_i[b]).wait()
        pltpu.make_async_copy(dst_hbm.at[pl.ds(0, C)], idst2.at[b],
                              sem_i[b]).wait()
        pltpu.make_async_copy(lg_hbm.at[pl.ds(0, C)], lbuf.at[b],
                              sem_i[b]).wait()

    def fire_gather(b4, b2):
        pltpu.async_copy(xv_hbm.at[isrc.at[b4]], vbuf.at[b2], sem_g[b2])

    def step(j, b2, b4):
        off = base + j * C

        @pl.when(j >= 1)
        def _():
            pltpu.make_async_copy(vbuf.at[b2 ^ 1], sharr.at[pl.ds(0, C)],
                                  sem_s[b2 ^ 1]).wait()

        @pl.when(j + 1 < NCH)
        def _():
            drain_idx((b4 + 1) % 4)
            fire_gather((b4 + 1) % 4, b2 ^ 1)

            @pl.when(j + 2 < NCH)
            def _():
                load_idx(j + 2, (b4 + 2) % 4)

        for g in range(GROUPS):
            sl = pl.ds(g * LANES, LANES)
            d = idst2[b4, sl]
            lg = lbuf[b4, sl]
            mv = plsc.load_gather(marrm, [d])
            ev = jnp.exp(lg - mv)
            ebuf[sl] = ev
            plsc.addupdate_scatter(darr.at[0], [d], ev)
        pltpu.make_async_copy(xv_hbm.at[pl.ds(0, C)], vbuf.at[b2],
                              sem_g[b2]).wait()

        for h in range(2):
            pltpu.sync_copy(vs_hbm.at[pl.ds(off + h * (C // 2), C // 2)],
                            vsbuf)

            def mul(i4, carry2, h=h):
                for u in range(4):
                    il = i4 * 4 + u
                    i = h * (C // 2) + il
                    eb = plsc.load_gather(ebuf,
                                          [jnp.broadcast_to(i, (LANES,))])
                    for dd in range(D // LANES):
                        cs = pl.ds(dd * LANES, LANES)
                        vbuf[b2, i, cs] = vbuf[b2, i, cs] * vsbuf[il, cs] * eb
                return carry2

            lax.fori_loop(0, C // 8, mul, 0)
        pltpu.async_copy(vbuf.at[b2], sharr.at[idst2.at[b4]], sem_s[b2],
                         add=True)

    load_idx(0, 0)
    load_idx(1, 1)
    drain_idx(0)
    fire_gather(0, 0)

    def outer(t, carry):
        for u in range(4):
            j = 4 * t + u
            step(j, u % 2, u)
        return carry

    lax.fori_loop(0, NCH // 4, outer, 0)
    jt = NCH - 1
    step(jt, jt % 2, jt % 4)
    pltpu.make_async_copy(vbuf.at[jt % 2], sharr.at[pl.ds(0, C)],
                          sem_s[jt % 2]).wait()

    pltpu.sync_copy(darr, pden_hbm.at[wid])
    plsc.subcore_barrier()

    @pl.when(sid < EXP_T)
    def _export():
        for p in range(EXP_R // ZR):
            row = sid * EXP_R + p * ZR
            pltpu.sync_copy(sharr.at[pl.ds(row, ZR)], vsbuf)
            pltpu.sync_copy(vsbuf, pnum_hbm.at[cid, pl.ds(row, ZR)])


def _tc_combine(pnum, pden, sw, vself):
    def body(pn_ref, pd_ref, sw_ref, vs_ref, o_ref):
        num = pn_ref[0] + pn_ref[1]
        den = jnp.sum(pd_ref[...], axis=1, keepdims=True) + sw_ref[...]
        o_ref[...] = (num + sw_ref[...] * vs_ref[...]) / den

    return pl.pallas_call(
        body,
        out_shape=jax.ShapeDtypeStruct((N, D), jnp.float32),
    )(pnum, pden, sw, vself)


def kernel(edge_index, node_features, edge_features, distances, Wq, Wkf,
           Wkg, Wvf, Wvg, Wvs, Wr1k, br1k, Wr2k, br2k, Wr1v, br1v, Wr2v,
           br2v):
    src3 = edge_index[0].astype(jnp.int32).reshape(NW, NCH, C)
    dst3 = edge_index[1].astype(jnp.int32).reshape(NW, NCH, C)
    kscale, vscale = _prep_scales(edge_features, distances, Wkg, Wr1k,
                                  br1k, Wr2k, br2k, Wvg, Wr1v, br1v, Wr2v,
                                  br2v)
    x = node_features
    for l in range(L):
        q, xk, xv, vself = _proj(x, Wq[l], Wkf[l], Wvf[l], Wvs[l])
        lg3, pmax = _make_sc_pass1()(src3, dst3, xk, q, kscale[l])
        m, sw = _tc_mreduce(pmax.T)
        pden, pnum = _make_sc_pass2()(src3.reshape(E), dst3.reshape(E),
                                      lg3.reshape(E), m.reshape(N), xv,
                                      vscale[l])
        x = _tc_combine(pnum, pden.reshape(NW, N).T, sw, vself)
    return x


# speedup vs baseline: 1.0443x; 1.0443x over previous
import functools
import math

import jax
import jax.numpy as jnp
from jax import lax
from jax.experimental import pallas as pl
from jax.experimental.pallas import tpu as pltpu
from jax.experimental.pallas import tpu_sc as plsc

NC = 2
NS = 16
NW = NC * NS
LANES = 16

N = 10000
E = 320000
D = 128
DK = 32
DE = 16
H = 64
L = 3

EW = E // NW
C = 80
NCH = EW // C
EXP_T = 10
EXP_R = N // EXP_T
ZR = 40
GROUPS = C // LANES

_MESH = dict(core_axis_name="c", subcore_axis_name="s", num_cores=NC,
             num_subcores=NS)


def _wid():
    return lax.axis_index("s") * NC + lax.axis_index("c")


def _prep_scales(ef, dist, Wkg, Wr1k, br1k, Wr2k, br2k, Wvg, Wr1v, br1v,
                 Wr2v, br2v):
    BE = 4000

    def body(ef_ref, dist_ref, Wkg_ref, Wr1k_ref, br1k_ref, Wr2k_ref,
             br2k_ref, Wvg_ref, Wr1v_ref, br1v_ref, Wr2v_ref, br2v_ref,
             ks_ref, vs_ref):
        efb = ef_ref[...]
        db = dist_ref[...]
        tk = jnp.tanh(db * Wr1k_ref[0] + br1k_ref[0])
        rk = jnp.dot(tk, Wr2k_ref[0], preferred_element_type=jnp.float32)
        rk = rk + br2k_ref[0]
        tv = jnp.tanh(db * Wr1v_ref[0] + br1v_ref[0])
        rv = jnp.dot(tv, Wr2v_ref[0], preferred_element_type=jnp.float32)
        rv = rv + br2v_ref[0]
        ks_ref[0] = jnp.dot(efb, Wkg_ref[0],
                            preferred_element_type=jnp.float32) * rk
        vs_ref[0] = jnp.dot(efb, Wvg_ref[0],
                            preferred_element_type=jnp.float32) * rv

    grid = (L, E // BE)
    out = pl.pallas_call(
        body,
        grid=grid,
        in_specs=[
            pl.BlockSpec((BE, DE), lambda l, b: (b, 0)),
            pl.BlockSpec((BE, 1), lambda l, b: (b, 0)),
            pl.BlockSpec((1, DE, DK), lambda l, b: (l, 0, 0)),
            pl.BlockSpec((1, 1, H), lambda l, b: (l, 0, 0)),
            pl.BlockSpec((1, 1, H), lambda l, b: (l, 0, 0)),
            pl.BlockSpec((1, H, DK), lambda l, b: (l, 0, 0)),
            pl.BlockSpec((1, 1, DK), lambda l, b: (l, 0, 0)),
            pl.BlockSpec((1, DE, D), lambda l, b: (l, 0, 0)),
            pl.BlockSpec((1, 1, H), lambda l, b: (l, 0, 0)),
            pl.BlockSpec((1, 1, H), lambda l, b: (l, 0, 0)),
            pl.BlockSpec((1, H, D), lambda l, b: (l, 0, 0)),
            pl.BlockSpec((1, 1, D), lambda l, b: (l, 0, 0)),
        ],
        out_specs=[
            pl.BlockSpec((1, BE, DK), lambda l, b: (l, b, 0)),
            pl.BlockSpec((1, BE, D), lambda l, b: (l, b, 0)),
        ],
        out_shape=[
            jax.ShapeDtypeStruct((L, E, DK), jnp.float32),
            jax.ShapeDtypeStruct((L, E, D), jnp.float32),
        ],
    )(ef, dist, Wkg, Wr1k, br1k[:, None, :], Wr2k, br2k[:, None, :],
      Wvg, Wr1v, br1v[:, None, :], Wr2v, br2v[:, None, :])
    return out


def _proj(x, Wq_l, Wkf_l, Wvf_l, Wvs_l):
    BN = 2000

    def body(x_ref, wq_ref, wkf_ref, wvf_ref, wvs_ref, q_ref, xk_ref,
             xv_ref, vs_ref):
        xb = x_ref[...]
        q_ref[...] = jnp.dot(xb, wq_ref[...],
                             preferred_element_type=jnp.float32)
        xk_ref[...] = jnp.dot(xb, wkf_ref[...],
                              preferred_element_type=jnp.float32)
        xv_ref[...] = jnp.dot(xb, wvf_ref[...],
                              preferred_element_type=jnp.float32)
        vs_ref[...] = jnp.dot(xb, wvs_ref[...],
                              preferred_element_type=jnp.float32)

    return pl.pallas_call(
        body,
        grid=(N // BN,),
        in_specs=[
            pl.BlockSpec((BN, D), lambda b: (b, 0)),
            pl.BlockSpec((D, DK), lambda b: (0, 0)),
            pl.BlockSpec((D, DK), lambda b: (0, 0)),
            pl.BlockSpec((D, D), lambda b: (0, 0)),
            pl.BlockSpec((D, D), lambda b: (0, 0)),
        ],
        out_specs=[
            pl.BlockSpec((BN, DK), lambda b: (b, 0)),
            pl.BlockSpec((BN, DK), lambda b: (b, 0)),
            pl.BlockSpec((BN, D), lambda b: (b, 0)),
            pl.BlockSpec((BN, D), lambda b: (b, 0)),
        ],
        out_shape=[
            jax.ShapeDtypeStruct((N, DK), jnp.float32),
            jax.ShapeDtypeStruct((N, DK), jnp.float32),
            jax.ShapeDtypeStruct((N, D), jnp.float32),
            jax.ShapeDtypeStruct((N, D), jnp.float32),
        ],
    )(x, Wq_l, Wkf_l, Wvf_l, Wvs_l)


@functools.cache
def _make_sc_pass1():
    return functools.partial(
        pl.kernel,
        out_type=[
            jax.ShapeDtypeStruct((NW, NCH, C), jnp.float32),
            jax.ShapeDtypeStruct((NW, N), jnp.float32),
        ],
        mesh=plsc.VectorSubcoreMesh(**_MESH),
        scratch_types=[
            pltpu.VMEM((N,), jnp.float32),
            pltpu.VMEM((NCH, C), jnp.int32),
            pltpu.VMEM((NCH, C), jnp.int32),
            pltpu.VMEM((2, C, DK), jnp.float32),
            pltpu.VMEM((2, C, DK), jnp.float32),
            pltpu.VMEM((2, C, DK), jnp.float32),
            pltpu.VMEM((2, C), jnp.float32),
            pltpu.VMEM((LANES,), jnp.int32),
            pltpu.VMEM((LANES,), jnp.float32),
            pltpu.SemaphoreType.DMA,
            pltpu.SemaphoreType.DMA,
            pltpu.SemaphoreType.DMA,
            pltpu.SemaphoreType.DMA,
        ],
        compiler_params=pltpu.CompilerParams(use_tc_tiling_on_sc=False,
                                             needs_layout_passes=False),
    )(_sc_pass1_body)


def _sc_pass1_body(src_hbm, dst_hbm, xk_hbm, q_hbm, ks_hbm,
                   lg_hbm, pmax_hbm,
                   marr, srcall, dstall, kbuf, qbuf, ksbuf, lgw, k16, v16,
                   sem_in0, sem_in1, sem_lg0, sem_lg1):
    wid = _wid()
    base = wid * EW
    sems = (sem_in0, sem_in1)
    sem_lg = (sem_lg0, sem_lg1)
    neg = jnp.full((LANES,), -1e30, dtype=jnp.float32)
    iota = lax.iota(jnp.int32, LANES)
    scale = 1.0 / math.sqrt(float(DK))

    pltpu.sync_copy(src_hbm.at[wid], srcall)
    pltpu.sync_copy(dst_hbm.at[wid], dstall)

    def init(i, carry):
        marr[pl.ds(i * LANES, LANES)] = neg
        return carry

    lax.fori_loop(0, N // LANES, init, 0)

    def fire(j, b):
        off = base + j * C
        pltpu.async_copy(xk_hbm.at[srcall.at[j]], kbuf.at[b], sems[b])
        pltpu.async_copy(q_hbm.at[dstall.at[j]], qbuf.at[b], sems[b])
        pltpu.async_copy(ks_hbm.at[pl.ds(off, C)], ksbuf.at[b], sems[b])

    def drain_in(b):
        pltpu.make_async_copy(xk_hbm.at[pl.ds(0, C)], kbuf.at[b],
                              sems[b]).wait()
        pltpu.make_async_copy(q_hbm.at[pl.ds(0, C)], qbuf.at[b],
                              sems[b]).wait()
        pltpu.make_async_copy(ks_hbm.at[pl.ds(0, C)], ksbuf.at[b],
                              sems[b]).wait()

    def compute(j, b):
        @pl.when(j >= 2)
        def _():
            pltpu.make_async_copy(lgw.at[b], lg_hbm.at[wid, 0],
                                  sem_lg[b]).wait()

        for g in range(GROUPS):
            rows = iota + (g * LANES)
            acc = jnp.zeros((LANES,), dtype=jnp.float32)
            for d in range(DK):
                cols = jnp.full((LANES,), d, dtype=jnp.int32)
                a = plsc.load_gather(kbuf.at[b], [rows, cols])
                kk = plsc.load_gather(ksbuf.at[b], [rows, cols])
                qq = plsc.load_gather(qbuf.at[b], [rows, cols])
                acc = acc + a * kk * qq
            val0 = acc * scale
            lgw[b, pl.ds(g * LANES, LANES)] = val0
            dvec = dstall[j, pl.ds(g * LANES, LANES)]
            ksrt, vsrt = plsc.sort_key_val(dvec, val0)
            k16[...] = ksrt
            val = vsrt
            for sh in (1, 2, 4, 8):
                v16[...] = val
                idx = jnp.maximum(iota - sh, 0)
                kprev = plsc.load_gather(k16, [idx])
                vprev = plsc.load_gather(v16, [idx])
                same = (kprev == ksrt) & (iota >= sh)
                val = jnp.where(same, jnp.maximum(val, vprev), val)
            knext = plsc.load_gather(k16, [jnp.minimum(iota + 1, LANES - 1)])
            is_last = (knext != ksrt) | (iota == LANES - 1)
            old = plsc.load_gather(marr, [ksrt])
            plsc.store_scatter(marr, [ksrt], jnp.maximum(old, val),
                               mask=is_last)
        pltpu.async_copy(lgw.at[b], lg_hbm.at[wid, j], sem_lg[b])

    fire(0, 0)

    def outer(t, carry):
        for b in range(2):
            j = 2 * t + b

            @pl.when(j + 1 < NCH)
            def _():
                fire(j + 1, b ^ 1)

            drain_in(b)
            compute(j, b)
        return carry

    lax.fori_loop(0, NCH // 2, outer, 0)
    jt = NCH - 1
    drain_in(jt % 2)
    compute(jt, jt % 2)
    for b in range(2):
        pltpu.make_async_copy(lgw.at[b], lg_hbm.at[wid, 0],
                              sem_lg[b]).wait()

    pltpu.sync_copy(marr, pmax_hbm.at[wid])


def _tc_mreduce(pmax):
    def body(p_ref, m_ref, sw_ref):
        m = jnp.max(p_ref[...], axis=1, keepdims=True)
        m = jnp.maximum(m, 1.0)
        m_ref[...] = m
        sw_ref[...] = jnp.exp(1.0 - m)

    return pl.pallas_call(
        body,
        out_shape=[
            jax.ShapeDtypeStruct((N, 1), jnp.float32),
            jax.ShapeDtypeStruct((N, 1), jnp.float32),
        ],
    )(pmax)


@functools.cache
def _make_sc_pass2():
    return functools.partial(
        pl.kernel,
        out_type=[
            jax.ShapeDtypeStruct((NW, 1, N), jnp.float32),
            jax.ShapeDtypeStruct((NC, N, D), jnp.float32),
        ],
        mesh=plsc.VectorSubcoreMesh(**_MESH),
        scratch_types=[
            pltpu.VMEM_SHARED((N, D), jnp.float32),
            pltpu.VMEM((N,), jnp.float32),
            pltpu.VMEM((1, N), jnp.float32),
            pltpu.VMEM((2, C), jnp.int32),
            pltpu.VMEM((2, C), jnp.int32),
            pltpu.VMEM((2, C), jnp.float32),
            pltpu.VMEM((C,), jnp.float32),
            pltpu.VMEM((C, D), jnp.float32),
            pltpu.VMEM((C, D), jnp.float32),
            pltpu.SemaphoreType.DMA,
            pltpu.SemaphoreType.DMA,
            pltpu.SemaphoreType.DMA,
        ],
        compiler_params=pltpu.CompilerParams(needs_layout_passes=False),
    )(_sc_pass2_body)


def _sc_pass2_body(src_hbm, dst_hbm, lg_hbm, m_hbm, xv_hbm, vs_hbm,
                   pden_hbm, pnum_hbm,
                   sharr, marrm, darr, isrc, idst2, lbuf, ebuf, vbuf,
                   vsbuf, sem, sem_i0, sem_i1):
    cid = lax.axis_index("c")
    sid = lax.axis_index("s")
    wid = _wid()
    base = wid * EW
    zero = jnp.zeros((LANES,), dtype=jnp.float32)

    pltpu.sync_copy(m_hbm, marrm)

    def zinit(i, carry):
        r = i // (D // LANES)
        col = (i % (D // LANES)) * LANES
        vsbuf[r, pl.ds(col, LANES)] = zero
        return carry

    lax.fori_loop(0, ZR * (D // LANES), zinit, 0)

    def dinit(i, carry):
        darr[0, pl.ds(i * LANES, LANES)] = zero
        return carry

    lax.fori_loop(0, N // LANES, dinit, 0)

    @pl.when(sid < EXP_T)
    def _zero():
        for p in range(EXP_R // ZR):
            pltpu.sync_copy(vsbuf.at[pl.ds(0, ZR)],
                            sharr.at[pl.ds(sid * EXP_R + p * ZR, ZR)])

    plsc.subcore_barrier()

    sem_i = (sem_i0, sem_i1)

    def load_idx(j, b):
        off = base + j * C
        pltpu.async_copy(src_hbm.at[pl.ds(off, C)], isrc.at[b], sem_i[b])
        pltpu.async_copy(dst_hbm.at[pl.ds(off, C)], idst2.at[b], sem_i[b])
        pltpu.async_copy(lg_hbm.at[pl.ds(off, C)], lbuf.at[b], sem_i[b])

    def drain_idx(b):
        pltpu.make_async_copy(src_hbm.at[pl.ds(0, C)], isrc.at[b],
                              sem_i[b]).wait()
        pltpu.make_async_copy(dst_hbm.at[pl.ds(0, C)], idst2.at[b],
                              sem_i[b]).wait()
        pltpu.make_async_copy(lg_hbm.at[pl.ds(0, C)], lbuf.at[b],
                              sem_i[b]).wait()

    def step(j, b):
        off = base + j * C
        drain_idx(b)
        cp = pltpu.async_copy(xv_hbm.at[isrc.at[b]], vbuf, sem)

        @pl.when(j + 1 < NCH)
        def _():
            load_idx(j + 1, b ^ 1)

        pltpu.sync_copy(vs_hbm.at[pl.ds(off, C)], vsbuf)
        for g in range(GROUPS):
            sl = pl.ds(g * LANES, LANES)
            d = idst2[b, sl]
            lg = lbuf[b, sl]
            mv = plsc.load_gather(marrm, [d])
            ev = jnp.exp(lg - mv)
            ebuf[sl] = ev
            plsc.addupdate_scatter(darr.at[0], [d], ev)
        cp.wait()

        def mul(i4, carry2):
            for u in range(4):
                i = i4 * 4 + u
                eb = plsc.load_gather(ebuf, [jnp.broadcast_to(i, (LANES,))])
                for dd in range(D // LANES):
                    cs = pl.ds(dd * LANES, LANES)
                    vbuf[i, cs] = vbuf[i, cs] * vsbuf[i, cs] * eb
            return carry2

        lax.fori_loop(0, C // 4, mul, 0)
        pltpu.sync_copy(vbuf, sharr.at[idst2.at[b]], add=True)

    load_idx(0, 0)

    def outer(t, carry):
        for b in range(2):
            step(2 * t + b, b)
        return carry

    lax.fori_loop(0, NCH // 2, outer, 0)
    step(NCH - 1, (NCH - 1) % 2)

    pltpu.sync_copy(darr, pden_hbm.at[wid])
    plsc.subcore_barrier()

    @pl.when(sid < EXP_T)
    def _export():
        for p in range(EXP_R // ZR):
            row = sid * EXP_R + p * ZR
            pltpu.sync_copy(sharr.at[pl.ds(row, ZR)], vsbuf.at[pl.ds(0, ZR)])
            pltpu.sync_copy(vsbuf.at[pl.ds(0, ZR)],
                            pnum_hbm.at[cid, pl.ds(row, ZR)])


def _tc_combine(pnum, pden, sw, vself):
    def body(pn_ref, pd_ref, sw_ref, vs_ref, o_ref):
        num = pn_ref[0] + pn_ref[1]
        den = jnp.sum(pd_ref[...], axis=1, keepdims=True) + sw_ref[...]
        o_ref[...] = (num + sw_ref[...] * vs_ref[...]) / den

    return pl.pallas_call(
        body,
        out_shape=jax.ShapeDtypeStruct((N, D), jnp.float32),
    )(pnum, pden, sw, vself)


def kernel(edge_index, node_features, edge_features, distances, Wq, Wkf,
           Wkg, Wvf, Wvg, Wvs, Wr1k, br1k, Wr2k, br2k, Wr1v, br1v, Wr2v,
           br2v):
    src3 = edge_index[0].astype(jnp.int32).reshape(NW, NCH, C)
    dst3 = edge_index[1].astype(jnp.int32).reshape(NW, NCH, C)
    kscale, vscale = _prep_scales(edge_features, distances, Wkg, Wr1k,
                                  br1k, Wr2k, br2k, Wvg, Wr1v, br1v, Wr2v,
                                  br2v)
    x = node_features
    for l in range(L):
        q, xk, xv, vself = _proj(x, Wq[l], Wkf[l], Wvf[l], Wvs[l])
        lg3, pmax = _make_sc_pass1()(src3, dst3, xk, q, kscale[l])
        m, sw = _tc_mreduce(pmax.T)
        pden, pnum = _make_sc_pass2()(src3.reshape(E), dst3.reshape(E),
                                      lg3.reshape(E), m.reshape(N), xv,
                                      vscale[l])
        x = _tc_combine(pnum, pden.reshape(NW, N).T, sw, vself)
    return x


# speedup vs baseline: 1.0461x; 1.0017x over previous
import functools
import math

import jax
import jax.numpy as jnp
from jax import lax
from jax.experimental import pallas as pl
from jax.experimental.pallas import tpu as pltpu
from jax.experimental.pallas import tpu_sc as plsc

NC = 2
NS = 16
NW = NC * NS
LANES = 16

N = 10000
E = 320000
D = 128
DK = 32
DE = 16
H = 64
L = 3

EW = E // NW
C = 80
NCH = EW // C
EXP_T = 10
EXP_R = N // EXP_T
ZR = 40
GROUPS = C // LANES

_MESH = dict(core_axis_name="c", subcore_axis_name="s", num_cores=NC,
             num_subcores=NS)


def _wid():
    return lax.axis_index("s") * NC + lax.axis_index("c")


def _prep_scales(ef, dist, Wkg, Wr1k, br1k, Wr2k, br2k, Wvg, Wr1v, br1v,
                 Wr2v, br2v):
    BE = 4000

    def body(ef_ref, dist_ref, Wkg_ref, Wr1k_ref, br1k_ref, Wr2k_ref,
             br2k_ref, Wvg_ref, Wr1v_ref, br1v_ref, Wr2v_ref, br2v_ref,
             ks_ref, vs_ref):
        efb = ef_ref[...]
        db = dist_ref[...]
        tk = jnp.tanh(db * Wr1k_ref[0] + br1k_ref[0])
        rk = jnp.dot(tk, Wr2k_ref[0], preferred_element_type=jnp.float32)
        rk = rk + br2k_ref[0]
        tv = jnp.tanh(db * Wr1v_ref[0] + br1v_ref[0])
        rv = jnp.dot(tv, Wr2v_ref[0], preferred_element_type=jnp.float32)
        rv = rv + br2v_ref[0]
        ks_ref[0] = jnp.dot(efb, Wkg_ref[0],
                            preferred_element_type=jnp.float32) * rk
        vs_ref[0] = jnp.dot(efb, Wvg_ref[0],
                            preferred_element_type=jnp.float32) * rv

    grid = (L, E // BE)
    out = pl.pallas_call(
        body,
        grid=grid,
        in_specs=[
            pl.BlockSpec((BE, DE), lambda l, b: (b, 0)),
            pl.BlockSpec((BE, 1), lambda l, b: (b, 0)),
            pl.BlockSpec((1, DE, DK), lambda l, b: (l, 0, 0)),
            pl.BlockSpec((1, 1, H), lambda l, b: (l, 0, 0)),
            pl.BlockSpec((1, 1, H), lambda l, b: (l, 0, 0)),
            pl.BlockSpec((1, H, DK), lambda l, b: (l, 0, 0)),
            pl.BlockSpec((1, 1, DK), lambda l, b: (l, 0, 0)),
            pl.BlockSpec((1, DE, D), lambda l, b: (l, 0, 0)),
            pl.BlockSpec((1, 1, H), lambda l, b: (l, 0, 0)),
            pl.BlockSpec((1, 1, H), lambda l, b: (l, 0, 0)),
            pl.BlockSpec((1, H, D), lambda l, b: (l, 0, 0)),
            pl.BlockSpec((1, 1, D), lambda l, b: (l, 0, 0)),
        ],
        out_specs=[
            pl.BlockSpec((1, BE, DK), lambda l, b: (l, b, 0)),
            pl.BlockSpec((1, BE, D), lambda l, b: (l, b, 0)),
        ],
        out_shape=[
            jax.ShapeDtypeStruct((L, E, DK), jnp.float32),
            jax.ShapeDtypeStruct((L, E, D), jnp.float32),
        ],
    )(ef, dist, Wkg, Wr1k, br1k[:, None, :], Wr2k, br2k[:, None, :],
      Wvg, Wr1v, br1v[:, None, :], Wr2v, br2v[:, None, :])
    return out


def _proj(x, Wq_l, Wkf_l, Wvf_l, Wvs_l):
    BN = 2000

    def body(x_ref, wq_ref, wkf_ref, wvf_ref, wvs_ref, q_ref, xk_ref,
             xv_ref, vs_ref):
        xb = x_ref[...]
        q_ref[...] = jnp.dot(xb, wq_ref[...],
                             preferred_element_type=jnp.float32)
        xk_ref[...] = jnp.dot(xb, wkf_ref[...],
                              preferred_element_type=jnp.float32)
        xv_ref[...] = jnp.dot(xb, wvf_ref[...],
                              preferred_element_type=jnp.float32)
        vs_ref[...] = jnp.dot(xb, wvs_ref[...],
                              preferred_element_type=jnp.float32)

    return pl.pallas_call(
        body,
        grid=(N // BN,),
        in_specs=[
            pl.BlockSpec((BN, D), lambda b: (b, 0)),
            pl.BlockSpec((D, DK), lambda b: (0, 0)),
            pl.BlockSpec((D, DK), lambda b: (0, 0)),
            pl.BlockSpec((D, D), lambda b: (0, 0)),
            pl.BlockSpec((D, D), lambda b: (0, 0)),
        ],
        out_specs=[
            pl.BlockSpec((BN, DK), lambda b: (b, 0)),
            pl.BlockSpec((BN, DK), lambda b: (b, 0)),
            pl.BlockSpec((BN, D), lambda b: (b, 0)),
            pl.BlockSpec((BN, D), lambda b: (b, 0)),
        ],
        out_shape=[
            jax.ShapeDtypeStruct((N, DK), jnp.float32),
            jax.ShapeDtypeStruct((N, DK), jnp.float32),
            jax.ShapeDtypeStruct((N, D), jnp.float32),
            jax.ShapeDtypeStruct((N, D), jnp.float32),
        ],
    )(x, Wq_l, Wkf_l, Wvf_l, Wvs_l)


@functools.cache
def _make_sc_pass1():
    return functools.partial(
        pl.kernel,
        out_type=[
            jax.ShapeDtypeStruct((NW, NCH, C), jnp.float32),
            jax.ShapeDtypeStruct((NW, N), jnp.float32),
        ],
        mesh=plsc.VectorSubcoreMesh(**_MESH),
        scratch_types=[
            pltpu.VMEM((N,), jnp.float32),
            pltpu.VMEM((NCH, C), jnp.int32),
            pltpu.VMEM((NCH, C), jnp.int32),
            pltpu.VMEM((2, C, DK), jnp.float32),
            pltpu.VMEM((2, C, DK), jnp.float32),
            pltpu.VMEM((2, C, DK), jnp.float32),
            pltpu.VMEM((2, C), jnp.float32),
            pltpu.VMEM((LANES,), jnp.int32),
            pltpu.VMEM((LANES,), jnp.float32),
            pltpu.SemaphoreType.DMA,
            pltpu.SemaphoreType.DMA,
            pltpu.SemaphoreType.DMA,
            pltpu.SemaphoreType.DMA,
        ],
        compiler_params=pltpu.CompilerParams(use_tc_tiling_on_sc=False,
                                             needs_layout_passes=False),
    )(_sc_pass1_body)


def _sc_pass1_body(src_hbm, dst_hbm, xk_hbm, q_hbm, ks_hbm,
                   lg_hbm, pmax_hbm,
                   marr, srcall, dstall, kbuf, qbuf, ksbuf, lgw, k16, v16,
                   sem_in0, sem_in1, sem_lg0, sem_lg1):
    wid = _wid()
    base = wid * EW
    sems = (sem_in0, sem_in1)
    sem_lg = (sem_lg0, sem_lg1)
    neg = jnp.full((LANES,), -1e30, dtype=jnp.float32)
    iota = lax.iota(jnp.int32, LANES)
    scale = 1.0 / math.sqrt(float(DK))

    pltpu.sync_copy(src_hbm.at[wid], srcall)
    pltpu.sync_copy(dst_hbm.at[wid], dstall)

    def init(i, carry):
        marr[pl.ds(i * LANES, LANES)] = neg
        return carry

    lax.fori_loop(0, N // LANES, init, 0)

    def fire(j, b):
        off = base + j * C
        pltpu.async_copy(xk_hbm.at[srcall.at[j]], kbuf.at[b], sems[b])
        pltpu.async_copy(q_hbm.at[dstall.at[j]], qbuf.at[b], sems[b])
        pltpu.async_copy(ks_hbm.at[pl.ds(off, C)], ksbuf.at[b], sems[b])

    def drain_in(b):
        pltpu.make_async_copy(xk_hbm.at[pl.ds(0, C)], kbuf.at[b],
                              sems[b]).wait()
        pltpu.make_async_copy(q_hbm.at[pl.ds(0, C)], qbuf.at[b],
                              sems[b]).wait()
        pltpu.make_async_copy(ks_hbm.at[pl.ds(0, C)], ksbuf.at[b],
                              sems[b]).wait()

    def compute(j, b):
        @pl.when(j >= 2)
        def _():
            pltpu.make_async_copy(lgw.at[b], lg_hbm.at[wid, 0],
                                  sem_lg[b]).wait()

        for g in range(GROUPS):
            rows = iota + (g * LANES)
            acc = jnp.zeros((LANES,), dtype=jnp.float32)
            for d in range(DK):
                cols = jnp.full((LANES,), d, dtype=jnp.int32)
                a = plsc.load_gather(kbuf.at[b], [rows, cols])
                kk = plsc.load_gather(ksbuf.at[b], [rows, cols])
                qq = plsc.load_gather(qbuf.at[b], [rows, cols])
                acc = acc + a * kk * qq
            val0 = acc * scale
            lgw[b, pl.ds(g * LANES, LANES)] = val0
            dvec = dstall[j, pl.ds(g * LANES, LANES)]
            ksrt, vsrt = plsc.sort_key_val(dvec, val0)
            k16[...] = ksrt
            val = vsrt
            for sh in (1, 2, 4, 8):
                v16[...] = val
                idx = jnp.maximum(iota - sh, 0)
                kprev = plsc.load_gather(k16, [idx])
                vprev = plsc.load_gather(v16, [idx])
                same = (kprev == ksrt) & (iota >= sh)
                val = jnp.where(same, jnp.maximum(val, vprev), val)
            knext = plsc.load_gather(k16, [jnp.minimum(iota + 1, LANES - 1)])
            is_last = (knext != ksrt) | (iota == LANES - 1)
            old = plsc.load_gather(marr, [ksrt])
            plsc.store_scatter(marr, [ksrt], jnp.maximum(old, val),
                               mask=is_last)
        pltpu.async_copy(lgw.at[b], lg_hbm.at[wid, j], sem_lg[b])

    fire(0, 0)

    def outer(t, carry):
        for b in range(2):
            j = 2 * t + b

            @pl.when(j + 1 < NCH)
            def _():
                fire(j + 1, b ^ 1)

            drain_in(b)
            compute(j, b)
        return carry

    lax.fori_loop(0, NCH // 2, outer, 0)
    jt = NCH - 1
    drain_in(jt % 2)
    compute(jt, jt % 2)
    for b in range(2):
        pltpu.make_async_copy(lgw.at[b], lg_hbm.at[wid, 0],
                              sem_lg[b]).wait()

    pltpu.sync_copy(marr, pmax_hbm.at[wid])


def _tc_mreduce(pmax):
    def body(p_ref, m_ref, sw_ref):
        m = jnp.max(p_ref[...], axis=1, keepdims=True)
        m = jnp.maximum(m, 1.0)
        m_ref[...] = m
        sw_ref[...] = jnp.exp(1.0 - m)

    return pl.pallas_call(
        body,
        out_shape=[
            jax.ShapeDtypeStruct((N, 1), jnp.float32),
            jax.ShapeDtypeStruct((N, 1), jnp.float32),
        ],
    )(pmax)


@functools.cache
def _make_sc_pass2():
    return functools.partial(
        pl.kernel,
        out_type=[
            jax.ShapeDtypeStruct((NW, 1, N), jnp.float32),
            jax.ShapeDtypeStruct((NC, N, D), jnp.float32),
        ],
        mesh=plsc.VectorSubcoreMesh(**_MESH),
        scratch_types=[
            pltpu.VMEM_SHARED((N, D), jnp.float32),
            pltpu.VMEM((N,), jnp.float32),
            pltpu.VMEM((1, N), jnp.float32),
            pltpu.VMEM((2, C), jnp.int32),
            pltpu.VMEM((2, C), jnp.int32),
            pltpu.VMEM((2, C), jnp.float32),
            pltpu.VMEM((C,), jnp.float32),
            pltpu.VMEM((C, D), jnp.float32),
            pltpu.VMEM((C, D), jnp.float32),
            pltpu.SemaphoreType.DMA,
            pltpu.SemaphoreType.DMA,
            pltpu.SemaphoreType.DMA,
        ],
        compiler_params=pltpu.CompilerParams(needs_layout_passes=False),
    )(_sc_pass2_body)


def _sc_pass2_body(src_hbm, dst_hbm, lg_hbm, m_hbm, xv_hbm, vs_hbm,
                   pden_hbm, pnum_hbm,
                   sharr, marrm, darr, isrc, idst2, lbuf, ebuf, vbuf,
                   vsbuf, sem, sem_i0, sem_i1):
    cid = lax.axis_index("c")
    sid = lax.axis_index("s")
    wid = _wid()
    base = wid * EW
    zero = jnp.zeros((LANES,), dtype=jnp.float32)

    pltpu.sync_copy(m_hbm, marrm)

    def zinit(i, carry):
        r = i // (D // LANES)
        col = (i % (D // LANES)) * LANES
        vsbuf[r, pl.ds(col, LANES)] = zero
        return carry

    lax.fori_loop(0, ZR * (D // LANES), zinit, 0)

    def dinit(i, carry):
        darr[0, pl.ds(i * LANES, LANES)] = zero
        return carry

    lax.fori_loop(0, N // LANES, dinit, 0)

    @pl.when(sid < EXP_T)
    def _zero():
        for p in range(EXP_R // ZR):
            pltpu.sync_copy(vsbuf.at[pl.ds(0, ZR)],
                            sharr.at[pl.ds(sid * EXP_R + p * ZR, ZR)])

    plsc.subcore_barrier()

    sem_i = (sem_i0, sem_i1)

    def load_idx(j, b):
        off = base + j * C
        pltpu.async_copy(src_hbm.at[pl.ds(off, C)], isrc.at[b], sem_i[b])
        pltpu.async_copy(dst_hbm.at[pl.ds(off, C)], idst2.at[b], sem_i[b])
        pltpu.async_copy(lg_hbm.at[pl.ds(off, C)], lbuf.at[b], sem_i[b])

    def drain_idx(b):
        pltpu.make_async_copy(src_hbm.at[pl.ds(0, C)], isrc.at[b],
                              sem_i[b]).wait()
        pltpu.make_async_copy(dst_hbm.at[pl.ds(0, C)], idst2.at[b],
                              sem_i[b]).wait()
        pltpu.make_async_copy(lg_hbm.at[pl.ds(0, C)], lbuf.at[b],
                              sem_i[b]).wait()

    def step(j, b):
        off = base + j * C
        drain_idx(b)
        cp = pltpu.async_copy(xv_hbm.at[isrc.at[b]], vbuf, sem)

        @pl.when(j + 1 < NCH)
        def _():
            load_idx(j + 1, b ^ 1)

        pltpu.sync_copy(vs_hbm.at[pl.ds(off, C)], vsbuf)
        for g in range(GROUPS):
            sl = pl.ds(g * LANES, LANES)
            d = idst2[b, sl]
            lg = lbuf[b, sl]
            mv = plsc.load_gather(marrm, [d])
            ev = jnp.exp(lg - mv)
            ebuf[sl] = ev
            plsc.addupdate_scatter(darr.at[0], [d], ev)
        cp.wait()

        def mul(i4, carry2):
            for u in range(4):
                i = i4 * 4 + u
                eb = plsc.load_gather(ebuf, [jnp.broadcast_to(i, (LANES,))])
                for dd in range(D // LANES):
                    cs = pl.ds(dd * LANES, LANES)
                    vbuf[i, cs] = vbuf[i, cs] * vsbuf[i, cs] * eb
            return carry2

        lax.fori_loop(0, C // 4, mul, 0)
        pltpu.sync_copy(vbuf, sharr.at[idst2.at[b]], add=True)

    load_idx(0, 0)

    def outer(t, carry):
        for b in range(2):
            step(2 * t + b, b)
        return carry

    lax.fori_loop(0, NCH // 2, outer, 0)
    step(NCH - 1, (NCH - 1) % 2)

    pltpu.sync_copy(darr, pden_hbm.at[wid])
    plsc.subcore_barrier()

    @pl.when(sid < EXP_T)
    def _export():
        for p in range(EXP_R // ZR):
            row = sid * EXP_R + p * ZR
            pltpu.sync_copy(sharr.at[pl.ds(row, ZR)], vsbuf.at[pl.ds(0, ZR)])
            pltpu.sync_copy(vsbuf.at[pl.ds(0, ZR)],
                            pnum_hbm.at[cid, pl.ds(row, ZR)])


def _tc_combine(pnum, pden, sw, vself):
    def body(pn_ref, pd_ref, sw_ref, vs_ref, o_ref):
        num = pn_ref[0] + pn_ref[1]
        den = jnp.sum(pd_ref[...], axis=1, keepdims=True) + sw_ref[...]
        o_ref[...] = (num + sw_ref[...] * vs_ref[...]) / den

    return pl.pallas_call(
        body,
        out_shape=jax.ShapeDtypeStruct((N, D), jnp.float32),
    )(pnum, pden, sw, vself)


def _tc_combine_proj(pnum, pden, sw, vself, Wq_n, Wkf_n, Wvf_n, Wvs_n):
    def body(pn_ref, pd_ref, sw_ref, vs_ref, wq_ref, wkf_ref, wvf_ref,
             wvs_ref, q_ref, xk_ref, xv_ref, vn_ref):
        num = pn_ref[0] + pn_ref[1]
        den = jnp.sum(pd_ref[...], axis=1, keepdims=True) + sw_ref[...]
        xb = (num + sw_ref[...] * vs_ref[...]) / den
        q_ref[...] = jnp.dot(xb, wq_ref[...],
                             preferred_element_type=jnp.float32)
        xk_ref[...] = jnp.dot(xb, wkf_ref[...],
                              preferred_element_type=jnp.float32)
        xv_ref[...] = jnp.dot(xb, wvf_ref[...],
                              preferred_element_type=jnp.float32)
        vn_ref[...] = jnp.dot(xb, wvs_ref[...],
                              preferred_element_type=jnp.float32)

    return pl.pallas_call(
        body,
        out_shape=[
            jax.ShapeDtypeStruct((N, DK), jnp.float32),
            jax.ShapeDtypeStruct((N, DK), jnp.float32),
            jax.ShapeDtypeStruct((N, D), jnp.float32),
            jax.ShapeDtypeStruct((N, D), jnp.float32),
        ],
    )(pnum, pden, sw, vself, Wq_n, Wkf_n, Wvf_n, Wvs_n)


def kernel(edge_index, node_features, edge_features, distances, Wq, Wkf,
           Wkg, Wvf, Wvg, Wvs, Wr1k, br1k, Wr2k, br2k, Wr1v, br1v, Wr2v,
           br2v):
    src3 = edge_index[0].astype(jnp.int32).reshape(NW, NCH, C)
    dst3 = edge_index[1].astype(jnp.int32).reshape(NW, NCH, C)
    kscale, vscale = _prep_scales(edge_features, distances, Wkg, Wr1k,
                                  br1k, Wr2k, br2k, Wvg, Wr1v, br1v, Wr2v,
                                  br2v)
    q, xk, xv, vself = _proj(node_features, Wq[0], Wkf[0], Wvf[0], Wvs[0])
    for l in range(L):
        lg3, pmax = _make_sc_pass1()(src3, dst3, xk, q, kscale[l])
        m, sw = _tc_mreduce(pmax.T)
        pden, pnum = _make_sc_pass2()(src3.reshape(E), dst3.reshape(E),
                                      lg3.reshape(E), m.reshape(N), xv,
                                      vscale[l])
        pden_t = pden.reshape(NW, N).T
        if l < L - 1:
            q, xk, xv, vself = _tc_combine_proj(pnum, pden_t, sw, vself,
                                                Wq[l + 1], Wkf[l + 1],
                                                Wvf[l + 1], Wvs[l + 1])
        else:
            x = _tc_combine(pnum, pden_t, sw, vself)
    return x
